# Initial kernel scaffold; baseline (speedup 1.0000x reference)
import jax
import jax.numpy as jnp
from jax.experimental import pallas as pl


def kernel(x, edge_index, W1, b1, W2, b2):
    raise NotImplementedError("write your pallas kernel here")


# speedup vs baseline: 32.3823x; 32.3823x over previous
import functools

import jax
import jax.numpy as jnp
from jax import lax
from jax.experimental import pallas as pl
from jax.experimental.pallas import tpu as pltpu
from jax.experimental.pallas import tpu_sc as plsc

_N = 10000
_E = 320000
_D = 128
_H = 64
_C = 16

_NC = 2
_NS = 16
_NW = _NC * _NS
_CH = 128
_J = 80
_EPW = _J * _CH
_EP = _NW * _EPW
_PAD_ROWS = 128
_NP = _N + _PAD_ROWS
_RSTRIDE = 624
_RCHUNK = 640

_MESH = plsc.VectorSubcoreMesh(core_axis_name="c", subcore_axis_name="s")


def _make_agg(F):
    @functools.partial(
        pl.kernel,
        out_type=jax.ShapeDtypeStruct((_NC, _N, F), jnp.float32),
        mesh=_MESH,
        scratch_types=[
            pltpu.VMEM((_J, _CH), jnp.int32),
            pltpu.VMEM((_J, _CH), jnp.int32),
            pltpu.VMEM((_CH, F), jnp.float32),
            pltpu.VMEM((_CH, F), jnp.float32),
            pltpu.VMEM_SHARED((_NP, F), jnp.float32),
            pltpu.SemaphoreType.DMA,
            pltpu.SemaphoreType.DMA,
        ],
        compiler_params=pltpu.CompilerParams(use_tc_tiling_on_sc=False),
    )
    def agg(m_hbm, srcp_hbm, dstp_hbm, out_hbm, sidx, didx, buf0, buf1,
            acc, sem0, sem1):
        c = lax.axis_index("c")
        s = lax.axis_index("s")
        w = c * _NS + s
        r0 = s * _RSTRIDE
        pltpu.sync_copy(m_hbm.at[pl.ds(r0, _RCHUNK)],
                        acc.at[pl.ds(r0, _RCHUNK)])
        pltpu.sync_copy(srcp_hbm.at[w], sidx)
        pltpu.sync_copy(dstp_hbm.at[w], didx)
        plsc.subcore_barrier()

        pltpu.make_async_copy(m_hbm.at[sidx.at[0]], buf0, sem0).start()

        def body(jj, carry):
            j0 = jj * 2
            pltpu.make_async_copy(m_hbm.at[sidx.at[j0]], buf0, sem0).wait()
            pltpu.make_async_copy(m_hbm.at[sidx.at[j0 + 1]], buf1, sem1).start()
            pltpu.sync_copy(buf0, acc.at[didx.at[j0]], add=True)
            pltpu.make_async_copy(m_hbm.at[sidx.at[j0 + 1]], buf1, sem1).wait()

            @pl.when(j0 + 2 < _J)
            def _():
                pltpu.make_async_copy(m_hbm.at[sidx.at[j0 + 2]], buf0,
                                      sem0).start()

            pltpu.sync_copy(buf1, acc.at[didx.at[j0 + 1]], add=True)
            return carry

        lax.fori_loop(0, _J // 2, body, 0)
        plsc.subcore_barrier()
        pltpu.sync_copy(acc.at[pl.ds(r0, _RCHUNK)],
                        out_hbm.at[c, pl.ds(r0, _RCHUNK)])

    return agg


@functools.partial(
    pl.kernel,
    out_type=jax.ShapeDtypeStruct((_NC, _N, 16), jnp.float32),
    mesh=_MESH,
    scratch_types=[
        pltpu.VMEM((_J, _CH), jnp.int32),
        pltpu.VMEM((_RCHUNK, 16), jnp.float32),
        pltpu.VMEM_SHARED((_NP, 16), jnp.float32),
    ],
    compiler_params=pltpu.CompilerParams(use_tc_tiling_on_sc=False),
)
def _deg_kernel(dstp_hbm, out_hbm, didx, ones_v, acc):
    c = lax.axis_index("c")
    s = lax.axis_index("s")
    w = c * _NS + s
    r0 = s * _RSTRIDE
    e0 = jnp.where(jnp.arange(16, dtype=jnp.int32) == 0,
                   jnp.float32(1.0), jnp.float32(0.0))

    def fill(i, carry):
        ones_v[i, :] = e0
        return carry

    lax.fori_loop(0, _RCHUNK, fill, 0)
    pltpu.sync_copy(ones_v, acc.at[pl.ds(r0, _RCHUNK)])
    pltpu.sync_copy(dstp_hbm.at[w], didx)
    plsc.subcore_barrier()

    def body(j, carry):
        pltpu.sync_copy(ones_v.at[pl.ds(0, _CH)], acc.at[didx.at[j]],
                        add=True)
        return carry

    lax.fori_loop(0, _J, body, 0)
    plsc.subcore_barrier()
    pltpu.sync_copy(acc.at[pl.ds(r0, _RCHUNK)],
                    out_hbm.at[c, pl.ds(r0, _RCHUNK)])


_agg64 = _make_agg(_H)
_agg16 = _make_agg(_C)

_BM = 1000


def _tc1_body(x_ref, w1_ref, d0_ref, d1_ref, m1_ref, dis_ref):
    deg = d0_ref[:, 0:1] + d1_ref[:, 0:1] - 1.0
    dis = lax.rsqrt(deg)
    h = jnp.dot(x_ref[...], w1_ref[...], preferred_element_type=jnp.float32)
    m1_ref[...] = h * dis
    dis_ref[...] = jnp.broadcast_to(dis, (_BM, 16))


def _tc1(x, W1, d0, d1):
    grid = (_N // _BM,)
    return pl.pallas_call(
        _tc1_body,
        grid=grid,
        in_specs=[
            pl.BlockSpec((_BM, _D), lambda i: (i, 0)),
            pl.BlockSpec((_D, _H), lambda i: (0, 0)),
            pl.BlockSpec((_BM, 16), lambda i: (i, 0)),
            pl.BlockSpec((_BM, 16), lambda i: (i, 0)),
        ],
        out_specs=[
            pl.BlockSpec((_BM, _H), lambda i: (i, 0)),
            pl.BlockSpec((_BM, 16), lambda i: (i, 0)),
        ],
        out_shape=[
            jax.ShapeDtypeStruct((_N, _H), jnp.float32),
            jax.ShapeDtypeStruct((_N, 16), jnp.float32),
        ],
    )(x, W1, d0, d1)


def _tc2_body(a0_ref, a1_ref, m1_ref, dis_ref, b1_ref, w2_ref, m2_ref):
    dis = dis_ref[:, 0:1]
    x2 = (a0_ref[...] + a1_ref[...] - m1_ref[...]) * dis + b1_ref[...]
    x2 = jnp.maximum(x2, 0.0)
    m2_ref[...] = jnp.dot(x2, w2_ref[...],
                          preferred_element_type=jnp.float32) * dis


def _tc2(a0, a1, m1, dis16, b1, W2):
    grid = (_N // _BM,)
    return pl.pallas_call(
        _tc2_body,
        grid=grid,
        in_specs=[
            pl.BlockSpec((_BM, _H), lambda i: (i, 0)),
            pl.BlockSpec((_BM, _H), lambda i: (i, 0)),
            pl.BlockSpec((_BM, _H), lambda i: (i, 0)),
            pl.BlockSpec((_BM, 16), lambda i: (i, 0)),
            pl.BlockSpec((1, _H), lambda i: (0, 0)),
            pl.BlockSpec((_H, _C), lambda i: (0, 0)),
        ],
        out_specs=pl.BlockSpec((_BM, _C), lambda i: (i, 0)),
        out_shape=jax.ShapeDtypeStruct((_N, _C), jnp.float32),
    )(a0, a1, m1, dis16, b1, W2)


def _tc3_body(c0_ref, c1_ref, m2_ref, dis_ref, b2_ref, out_ref):
    dis = dis_ref[:, 0:1]
    out_ref[...] = ((c0_ref[...] + c1_ref[...] - m2_ref[...]) * dis
                    + b2_ref[...])


def _tc3(c0, c1, m2, dis16, b2):
    grid = (_N // _BM,)
    return pl.pallas_call(
        _tc3_body,
        grid=grid,
        in_specs=[
            pl.BlockSpec((_BM, _C), lambda i: (i, 0)),
            pl.BlockSpec((_BM, _C), lambda i: (i, 0)),
            pl.BlockSpec((_BM, _C), lambda i: (i, 0)),
            pl.BlockSpec((_BM, 16), lambda i: (i, 0)),
            pl.BlockSpec((1, _C), lambda i: (0, 0)),
        ],
        out_specs=pl.BlockSpec((_BM, _C), lambda i: (i, 0)),
        out_shape=jax.ShapeDtypeStruct((_N, _C), jnp.float32),
    )(c0, c1, m2, dis16, b2)


def kernel(x, edge_index, W1, b1, W2, b2):
    src = edge_index[0].astype(jnp.int32)
    dst = edge_index[1].astype(jnp.int32)
    pad = _EP - _E
    pad_i = jnp.arange(pad, dtype=jnp.int32)
    pad_src = (pad_i * 37) % _N
    pad_dst = _N + (pad_i % _PAD_ROWS)
    srcp = jnp.concatenate([src, pad_src]).reshape(_NW, _J, _CH)
    dstp = jnp.concatenate([dst, pad_dst]).reshape(_NW, _J, _CH)

    degp = _deg_kernel(dstp)
    m1, dis16 = _tc1(x, W1, degp[0], degp[1])
    ap1 = _agg64(m1, srcp, dstp)
    m2 = _tc2(ap1[0], ap1[1], m1, dis16, b1.reshape(1, _H), W2)
    ap2 = _agg16(m2, srcp, dstp)
    return _tc3(ap2[0], ap2[1], m2, dis16, b2.reshape(1, _C))


# speedup vs baseline: 42.7739x; 1.3209x over previous
import functools

import jax
import jax.numpy as jnp
from jax import lax
from jax.experimental import pallas as pl
from jax.experimental.pallas import tpu as pltpu
from jax.experimental.pallas import tpu_sc as plsc

_N = 10000
_E = 320000
_D = 128
_H = 64
_C = 16

_NC = 2
_NS = 16
_NW = _NC * _NS
_CH = 128
_J = 80
_EPW = _J * _CH
_EP = _NW * _EPW
_PAD_ROWS = 128
_NP = _N + _PAD_ROWS
_RSTRIDE = 624
_RCHUNK = 640

_MESH = plsc.VectorSubcoreMesh(core_axis_name="c", subcore_axis_name="s")


def _make_agg(F, G):
    ngroups = _J // G
    assert ngroups * G == _J and ngroups >= 2

    @functools.partial(
        pl.kernel,
        out_type=jax.ShapeDtypeStruct((_NC, _N, F), jnp.float32),
        mesh=_MESH,
        scratch_types=[
            pltpu.VMEM((_J, _CH), jnp.int32),
            pltpu.VMEM((_J, _CH), jnp.int32),
            pltpu.VMEM((2 * G, _CH, F), jnp.float32),
            pltpu.VMEM_SHARED((_NP, F), jnp.float32),
            pltpu.SemaphoreType.DMA,
            pltpu.SemaphoreType.DMA,
            pltpu.SemaphoreType.DMA,
            pltpu.SemaphoreType.DMA,
        ],
        compiler_params=pltpu.CompilerParams(use_tc_tiling_on_sc=False),
    )
    def agg(m_hbm, srcp_hbm, dstp_hbm, out_hbm, sidx, didx, bufs,
            acc, semg0, semg1, sems0, sems1):
        c = lax.axis_index("c")
        s = lax.axis_index("s")
        w = c * _NS + s
        r0 = s * _RSTRIDE
        semg = (semg0, semg1)
        sems = (sems0, sems1)
        pltpu.sync_copy(m_hbm.at[pl.ds(r0, _RCHUNK)],
                        acc.at[pl.ds(r0, _RCHUNK)])
        pltpu.sync_copy(srcp_hbm.at[w], sidx)
        pltpu.sync_copy(dstp_hbm.at[w], didx)
        plsc.subcore_barrier()

        def fire_gathers(g, h):
            for i in range(G):
                pltpu.make_async_copy(m_hbm.at[sidx.at[g * G + i]],
                                      bufs.at[h * G + i], semg[h]).start()

        def drain_gathers(h):
            for i in range(G):
                pltpu.make_async_copy(m_hbm.at[sidx.at[i]],
                                      bufs.at[h * G + i], semg[h]).wait()

        def fire_scatters(g, h):
            for i in range(G):
                pltpu.async_copy(bufs.at[h * G + i],
                                 acc.at[didx.at[g * G + i]], sems[h],
                                 add=True)

        def drain_scatters(h):
            for i in range(G):
                pltpu.make_async_copy(bufs.at[h * G + i],
                                      acc.at[didx.at[i]], sems[h]).wait()

        fire_gathers(0, 0)
        for g in range(ngroups):
            h = g % 2
            oth = 1 - h
            if g >= 1:
                drain_scatters(oth)
            if g + 1 < ngroups:
                fire_gathers(g + 1, oth)
            drain_gathers(h)
            fire_scatters(g, h)
        drain_scatters((ngroups - 1) % 2)

        plsc.subcore_barrier()
        pltpu.sync_copy(acc.at[pl.ds(r0, _RCHUNK)],
                        out_hbm.at[c, pl.ds(r0, _RCHUNK)])

    return agg


@functools.partial(
    pl.kernel,
    out_type=jax.ShapeDtypeStruct((_NC, _N, 16), jnp.float32),
    mesh=_MESH,
    scratch_types=[
        pltpu.VMEM((_J, _CH), jnp.int32),
        pltpu.VMEM((_RCHUNK, 16), jnp.float32),
        pltpu.VMEM_SHARED((_NP, 16), jnp.float32),
        pltpu.SemaphoreType.DMA,
    ],
    compiler_params=pltpu.CompilerParams(use_tc_tiling_on_sc=False),
)
def _deg_kernel(dstp_hbm, out_hbm, didx, ones_v, acc, sem):
    c = lax.axis_index("c")
    s = lax.axis_index("s")
    w = c * _NS + s
    r0 = s * _RSTRIDE
    e0 = jnp.where(jnp.arange(16, dtype=jnp.int32) == 0,
                   jnp.float32(1.0), jnp.float32(0.0))

    def fill(i, carry):
        ones_v[i, :] = e0
        return carry

    lax.fori_loop(0, _RCHUNK, fill, 0)
    pltpu.sync_copy(ones_v, acc.at[pl.ds(r0, _RCHUNK)])
    pltpu.sync_copy(dstp_hbm.at[w], didx)
    plsc.subcore_barrier()

    for j in range(_J):
        pltpu.async_copy(ones_v.at[pl.ds(0, _CH)], acc.at[didx.at[j]],
                         sem, add=True)
    for j in range(_J):
        pltpu.make_async_copy(ones_v.at[pl.ds(0, _CH)], acc.at[didx.at[0]],
                              sem).wait()
    plsc.subcore_barrier()
    pltpu.sync_copy(acc.at[pl.ds(r0, _RCHUNK)],
                    out_hbm.at[c, pl.ds(r0, _RCHUNK)])


_agg64 = _make_agg(_H, 4)
_agg16 = _make_agg(_C, 8)

_BM = 1000


def _tc1_body(x_ref, w1_ref, d0_ref, d1_ref, m1_ref, dis_ref):
    deg = d0_ref[:, 0:1] + d1_ref[:, 0:1] - 1.0
    dis = lax.rsqrt(deg)
    h = jnp.dot(x_ref[...], w1_ref[...], preferred_element_type=jnp.float32)
    m1_ref[...] = h * dis
    dis_ref[...] = jnp.broadcast_to(dis, (_BM, 16))


def _tc1(x, W1, d0, d1):
    grid = (_N // _BM,)
    return pl.pallas_call(
        _tc1_body,
        grid=grid,
        in_specs=[
            pl.BlockSpec((_BM, _D), lambda i: (i, 0)),
            pl.BlockSpec((_D, _H), lambda i: (0, 0)),
            pl.BlockSpec((_BM, 16), lambda i: (i, 0)),
            pl.BlockSpec((_BM, 16), lambda i: (i, 0)),
        ],
        out_specs=[
            pl.BlockSpec((_BM, _H), lambda i: (i, 0)),
            pl.BlockSpec((_BM, 16), lambda i: (i, 0)),
        ],
        out_shape=[
            jax.ShapeDtypeStruct((_N, _H), jnp.float32),
            jax.ShapeDtypeStruct((_N, 16), jnp.float32),
        ],
    )(x, W1, d0, d1)


def _tc2_body(a0_ref, a1_ref, m1_ref, dis_ref, b1_ref, w2_ref, m2_ref):
    dis = dis_ref[:, 0:1]
    x2 = (a0_ref[...] + a1_ref[...] - m1_ref[...]) * dis + b1_ref[...]
    x2 = jnp.maximum(x2, 0.0)
    m2_ref[...] = jnp.dot(x2, w2_ref[...],
                          preferred_element_type=jnp.float32) * dis


def _tc2(a0, a1, m1, dis16, b1, W2):
    grid = (_N // _BM,)
    return pl.pallas_call(
        _tc2_body,
        grid=grid,
        in_specs=[
            pl.BlockSpec((_BM, _H), lambda i: (i, 0)),
            pl.BlockSpec((_BM, _H), lambda i: (i, 0)),
            pl.BlockSpec((_BM, _H), lambda i: (i, 0)),
            pl.BlockSpec((_BM, 16), lambda i: (i, 0)),
            pl.BlockSpec((1, _H), lambda i: (0, 0)),
            pl.BlockSpec((_H, _C), lambda i: (0, 0)),
        ],
        out_specs=pl.BlockSpec((_BM, _C), lambda i: (i, 0)),
        out_shape=jax.ShapeDtypeStruct((_N, _C), jnp.float32),
    )(a0, a1, m1, dis16, b1, W2)


def _tc3_body(c0_ref, c1_ref, m2_ref, dis_ref, b2_ref, out_ref):
    dis = dis_ref[:, 0:1]
    out_ref[...] = ((c0_ref[...] + c1_ref[...] - m2_ref[...]) * dis
                    + b2_ref[...])


def _tc3(c0, c1, m2, dis16, b2):
    grid = (_N // _BM,)
    return pl.pallas_call(
        _tc3_body,
        grid=grid,
        in_specs=[
            pl.BlockSpec((_BM, _C), lambda i: (i, 0)),
            pl.BlockSpec((_BM, _C), lambda i: (i, 0)),
            pl.BlockSpec((_BM, _C), lambda i: (i, 0)),
            pl.BlockSpec((_BM, 16), lambda i: (i, 0)),
            pl.BlockSpec((1, _C), lambda i: (0, 0)),
        ],
        out_specs=pl.BlockSpec((_BM, _C), lambda i: (i, 0)),
        out_shape=jax.ShapeDtypeStruct((_N, _C), jnp.float32),
    )(c0, c1, m2, dis16, b2)


def kernel(x, edge_index, W1, b1, W2, b2):
    src = edge_index[0].astype(jnp.int32)
    dst = edge_index[1].astype(jnp.int32)
    pad = _EP - _E
    pad_i = jnp.arange(pad, dtype=jnp.int32)
    pad_src = (pad_i * 37) % _N
    pad_dst = _N + (pad_i % _PAD_ROWS)
    srcp = jnp.concatenate([src, pad_src]).reshape(_NW, _J, _CH)
    dstp = jnp.concatenate([dst, pad_dst]).reshape(_NW, _J, _CH)

    degp = _deg_kernel(dstp)
    m1, dis16 = _tc1(x, W1, degp[0], degp[1])
    ap1 = _agg64(m1, srcp, dstp)
    m2 = _tc2(ap1[0], ap1[1], m1, dis16, b1.reshape(1, _H), W2)
    ap2 = _agg16(m2, srcp, dstp)
    return _tc3(ap2[0], ap2[1], m2, dis16, b2.reshape(1, _C))


# speedup vs baseline: 48.3341x; 1.1300x over previous
import functools

import jax
import jax.numpy as jnp
import numpy as np
from jax import lax
from jax.experimental import pallas as pl
from jax.experimental.pallas import tpu as pltpu
from jax.experimental.pallas import tpu_sc as plsc

_N = 10000
_E = 320000
_D = 128
_H = 64
_C = 16

_NC = 2
_NS = 16
_NW = _NC * _NS
_CH = 128
_J = 80
_EPW = _J * _CH
_EP = _NW * _EPW
_PAD_ROWS = 128
_NP = _N + _PAD_ROWS
_RSTRIDE = 624
_RCHUNK = 640

_MESH = plsc.VectorSubcoreMesh(core_axis_name="c", subcore_axis_name="s")


def _make_agg(F, G):
    ngroups = _J // G
    assert ngroups * G == _J and ngroups >= 2

    @functools.partial(
        pl.kernel,
        out_type=[jax.ShapeDtypeStruct((_N, F), jnp.float32),
                  jax.ShapeDtypeStruct((_N, F), jnp.float32)],
        mesh=_MESH,
        scratch_types=[
            pltpu.VMEM((_J, _CH), jnp.int32),
            pltpu.VMEM((_J, _CH), jnp.int32),
            pltpu.VMEM((2 * G, _CH, F), jnp.float32),
            pltpu.VMEM_SHARED((_NP, F), jnp.float32),
            pltpu.SemaphoreType.DMA,
            pltpu.SemaphoreType.DMA,
            pltpu.SemaphoreType.DMA,
            pltpu.SemaphoreType.DMA,
        ],
        compiler_params=pltpu.CompilerParams(use_tc_tiling_on_sc=False),
    )
    def agg(m_hbm, srcp_hbm, dstp_hbm, out0_hbm, out1_hbm, sidx, didx, bufs,
            acc, semg0, semg1, sems0, sems1):
        c = lax.axis_index("c")
        s = lax.axis_index("s")
        w = c * _NS + s
        r0 = s * _RSTRIDE
        semg = (semg0, semg1)
        sems = (sems0, sems1)
        pltpu.sync_copy(m_hbm.at[pl.ds(r0, _RCHUNK)],
                        acc.at[pl.ds(r0, _RCHUNK)])
        pltpu.sync_copy(srcp_hbm.at[w], sidx)
        pltpu.sync_copy(dstp_hbm.at[w], didx)
        plsc.subcore_barrier()

        def fire_gathers(g, h):
            for i in range(G):
                pltpu.make_async_copy(m_hbm.at[sidx.at[g * G + i]],
                                      bufs.at[h * G + i], semg[h]).start()

        def drain_gathers(h):
            for i in range(G):
                pltpu.make_async_copy(m_hbm.at[sidx.at[i]],
                                      bufs.at[h * G + i], semg[h]).wait()

        def fire_scatters(g, h):
            for i in range(G):
                pltpu.async_copy(bufs.at[h * G + i],
                                 acc.at[didx.at[g * G + i]], sems[h],
                                 add=True)

        def drain_scatters(h):
            for i in range(G):
                pltpu.make_async_copy(bufs.at[h * G + i],
                                      acc.at[didx.at[i]], sems[h]).wait()

        fire_gathers(0, 0)
        for g in range(ngroups):
            h = g % 2
            oth = 1 - h
            if g >= 1:
                drain_scatters(oth)
            if g + 1 < ngroups:
                fire_gathers(g + 1, oth)
            drain_gathers(h)
            fire_scatters(g, h)
        drain_scatters((ngroups - 1) % 2)

        plsc.subcore_barrier()
        @pl.when(c == 0)
        def _():
            pltpu.sync_copy(acc.at[pl.ds(r0, _RCHUNK)],
                            out0_hbm.at[pl.ds(r0, _RCHUNK)])

        @pl.when(c == 1)
        def _():
            pltpu.sync_copy(acc.at[pl.ds(r0, _RCHUNK)],
                            out1_hbm.at[pl.ds(r0, _RCHUNK)])

    return agg


@functools.partial(
    pl.kernel,
    out_type=[jax.ShapeDtypeStruct((_N, 16), jnp.float32),
              jax.ShapeDtypeStruct((_N, 16), jnp.float32)],
    mesh=_MESH,
    scratch_types=[
        pltpu.VMEM((_J, _CH), jnp.int32),
        pltpu.VMEM((_RCHUNK, 16), jnp.float32),
        pltpu.VMEM_SHARED((_NP, 16), jnp.float32),
        pltpu.SemaphoreType.DMA,
    ],
    compiler_params=pltpu.CompilerParams(use_tc_tiling_on_sc=False),
)
def _deg_kernel(dstp_hbm, out0_hbm, out1_hbm, didx, ones_v, acc, sem):
    c = lax.axis_index("c")
    s = lax.axis_index("s")
    w = c * _NS + s
    r0 = s * _RSTRIDE
    e0 = jnp.where(jnp.arange(16, dtype=jnp.int32) == 0,
                   jnp.float32(1.0), jnp.float32(0.0))

    def fill(i, carry):
        ones_v[i, :] = e0
        return carry

    lax.fori_loop(0, _RCHUNK, fill, 0)
    pltpu.sync_copy(ones_v, acc.at[pl.ds(r0, _RCHUNK)])
    pltpu.sync_copy(dstp_hbm.at[w], didx)
    plsc.subcore_barrier()

    for j in range(_J):
        pltpu.async_copy(ones_v.at[pl.ds(0, _CH)], acc.at[didx.at[j]],
                         sem, add=True)
    for j in range(_J):
        pltpu.make_async_copy(ones_v.at[pl.ds(0, _CH)], acc.at[didx.at[0]],
                              sem).wait()
    plsc.subcore_barrier()

    @pl.when(c == 0)
    def _():
        pltpu.sync_copy(acc.at[pl.ds(r0, _RCHUNK)],
                        out0_hbm.at[pl.ds(r0, _RCHUNK)])

    @pl.when(c == 1)
    def _():
        pltpu.sync_copy(acc.at[pl.ds(r0, _RCHUNK)],
                        out1_hbm.at[pl.ds(r0, _RCHUNK)])


_agg64 = _make_agg(_H, 4)
_agg16 = _make_agg(_C, 10)

_BM = 2000


def _tch_body(x_ref, w1_ref, h_ref):
    h_ref[...] = jnp.dot(x_ref[...], w1_ref[...],
                         preferred_element_type=jnp.float32)


def _tch(x, W1):
    grid = (_N // _BM,)
    return pl.pallas_call(
        _tch_body,
        grid=grid,
        in_specs=[
            pl.BlockSpec((_BM, _D), lambda i: (i, 0)),
            pl.BlockSpec((_D, _H), lambda i: (0, 0)),
        ],
        out_specs=pl.BlockSpec((_BM, _H), lambda i: (i, 0)),
        out_shape=jax.ShapeDtypeStruct((_N, _H), jnp.float32),
    )(x, W1)


def _tcm_body(h_ref, d0_ref, d1_ref, m1_ref, dis_ref):
    deg = d0_ref[:, 0:1] + d1_ref[:, 0:1] - 1.0
    dis = lax.rsqrt(deg)
    m1_ref[...] = h_ref[...] * dis
    dis_ref[...] = jnp.broadcast_to(dis, (_BM, 16))


def _tcm(h, d0, d1):
    grid = (_N // _BM,)
    return pl.pallas_call(
        _tcm_body,
        grid=grid,
        in_specs=[
            pl.BlockSpec((_BM, _H), lambda i: (i, 0)),
            pl.BlockSpec((_BM, 16), lambda i: (i, 0)),
            pl.BlockSpec((_BM, 16), lambda i: (i, 0)),
        ],
        out_specs=[
            pl.BlockSpec((_BM, _H), lambda i: (i, 0)),
            pl.BlockSpec((_BM, 16), lambda i: (i, 0)),
        ],
        out_shape=[
            jax.ShapeDtypeStruct((_N, _H), jnp.float32),
            jax.ShapeDtypeStruct((_N, 16), jnp.float32),
        ],
    )(h, d0, d1)


def _tc2_body(a0_ref, a1_ref, m1_ref, dis_ref, b1_ref, w2_ref, m2_ref):
    dis = dis_ref[:, 0:1]
    x2 = (a0_ref[...] + a1_ref[...] - m1_ref[...]) * dis + b1_ref[...]
    x2 = jnp.maximum(x2, 0.0)
    m2_ref[...] = jnp.dot(x2, w2_ref[...],
                          preferred_element_type=jnp.float32) * dis


def _tc2(a0, a1, m1, dis16, b1, W2):
    grid = (_N // _BM,)
    return pl.pallas_call(
        _tc2_body,
        grid=grid,
        in_specs=[
            pl.BlockSpec((_BM, _H), lambda i: (i, 0)),
            pl.BlockSpec((_BM, _H), lambda i: (i, 0)),
            pl.BlockSpec((_BM, _H), lambda i: (i, 0)),
            pl.BlockSpec((_BM, 16), lambda i: (i, 0)),
            pl.BlockSpec((1, _H), lambda i: (0, 0)),
            pl.BlockSpec((_H, _C), lambda i: (0, 0)),
        ],
        out_specs=pl.BlockSpec((_BM, _C), lambda i: (i, 0)),
        out_shape=jax.ShapeDtypeStruct((_N, _C), jnp.float32),
    )(a0, a1, m1, dis16, b1, W2)


def _tc3_body(c0_ref, c1_ref, m2_ref, dis_ref, b2_ref, out_ref):
    dis = dis_ref[:, 0:1]
    out_ref[...] = ((c0_ref[...] + c1_ref[...] - m2_ref[...]) * dis
                    + b2_ref[...])


def _tc3(c0, c1, m2, dis16, b2):
    grid = (_N // _BM,)
    return pl.pallas_call(
        _tc3_body,
        grid=grid,
        in_specs=[
            pl.BlockSpec((_BM, _C), lambda i: (i, 0)),
            pl.BlockSpec((_BM, _C), lambda i: (i, 0)),
            pl.BlockSpec((_BM, _C), lambda i: (i, 0)),
            pl.BlockSpec((_BM, 16), lambda i: (i, 0)),
            pl.BlockSpec((1, _C), lambda i: (0, 0)),
        ],
        out_specs=pl.BlockSpec((_BM, _C), lambda i: (i, 0)),
        out_shape=jax.ShapeDtypeStruct((_N, _C), jnp.float32),
    )(c0, c1, m2, dis16, b2)


_PAD = _EP - _E
_PAD_I = np.arange(_PAD, dtype=np.int32)
_PAD_SRC = np.asarray((_PAD_I * 37) % _N, dtype=np.int32)
_PAD_DST = np.asarray(_N + (_PAD_I % _PAD_ROWS), dtype=np.int32)


def kernel(x, edge_index, W1, b1, W2, b2):
    src = edge_index[0]
    dst = edge_index[1]
    srcp = jnp.concatenate([src, _PAD_SRC]).reshape(_NW, _J, _CH)
    dstp = jnp.concatenate([dst, _PAD_DST]).reshape(_NW, _J, _CH)

    h1 = _tch(x, W1)
    d0, d1 = _deg_kernel(dstp)
    m1, dis16 = _tcm(h1, d0, d1)
    a0, a1 = _agg64(m1, srcp, dstp)
    m2 = _tc2(a0, a1, m1, dis16, b1.reshape(1, _H), W2)
    c0, c1 = _agg16(m2, srcp, dstp)
    return _tc3(c0, c1, m2, dis16, b2.reshape(1, _C))
